# baseline (speedup 1.0000x reference)
import jax
import jax.numpy as jnp
from jax.experimental import pallas as pl
from jax.experimental.pallas import tpu as pltpu

_N, _DIN, _DOUT, _H, _K = 2048, 2048, 2048, 16384, 32

_BH_ENC = 512
_BN_TOP = 128
_BH_DEC = 512
_BN_FIN = 256
_BISECT_ITERS = 36


def _enc_body(x_ref, w_ref, b_ref, out_ref):
    out_ref[...] = jax.lax.dot_general(
        x_ref[...], w_ref[...], (((1,), (1,)), ((), ())),
        preferred_element_type=jnp.float32) + b_ref[...]


def _topk_body(pre_ref, hid_ref, thr_ref, cnt_ref):
    v = pre_ref[...]
    kf = jnp.float32(_K)

    cm = v[:, 0:128]
    for c in range(1, _H // 128):
        cm = jnp.maximum(cm, v[:, c * 128:(c + 1) * 128])
    hi0 = jnp.max(cm, axis=1, keepdims=True)
    lo0 = jnp.min(cm, axis=1, keepdims=True)

    def cbody(_, carry):
        lo, hi = carry
        mid = (lo + hi) * 0.5
        c = jnp.sum((cm >= mid).astype(jnp.float32), axis=1, keepdims=True)
        ge = c >= kf
        return jnp.where(ge, mid, lo), jnp.where(ge, hi, mid)

    clo, _ = jax.lax.fori_loop(0, 20, cbody, (lo0, hi0))

    def cond(state):
        i, lo, hi, cnt = state
        return jnp.logical_and(i < _BISECT_ITERS,
                               jnp.logical_not(jnp.all(cnt == kf)))

    def body(state):
        i, lo, hi, cnt = state
        for _ in range(4):
            mid = (lo + hi) * 0.5
            c = jnp.sum((v >= mid).astype(jnp.float32), axis=1, keepdims=True)
            ge = c >= kf
            lo = jnp.where(ge, mid, lo)
            hi = jnp.where(ge, hi, mid)
            cnt = jnp.where(ge, c, cnt)
        return (i + 4, lo, hi, cnt)

    _, lo, hi, _ = jax.lax.while_loop(
        cond, body, (jnp.int32(0), clo, hi0, jnp.full_like(hi0, 1e9)))
    t = lo
    hid = jnp.where(v >= t, jnp.maximum(v, 0.0), 0.0)
    hid_ref[...] = hid
    thr_ref[...] = t
    cnt_ref[...] = jnp.sum((hid > 0).astype(jnp.float32), axis=1,
                           keepdims=True)


def _dec_body(hid_ref, w_ref, out_ref):
    h = pl.program_id(0)

    @pl.when(h == 0)
    def _():
        out_ref[...] = jnp.zeros_like(out_ref)

    out_ref[...] += jax.lax.dot_general(
        hid_ref[...], w_ref[...], (((1,), (1,)), ((), ())),
        preferred_element_type=jnp.float32)


def _fin_body(sp_ref, x_ref, ws_ref, bd_ref, bs_ref, y_ref, cnt_ref,
              pred_ref, ls_ref, ps_ref):
    skip = jax.lax.dot_general(
        x_ref[...], ws_ref[...], (((1,), (1,)), ((), ())),
        preferred_element_type=jnp.float32)
    pred = sp_ref[...] + skip + bd_ref[...] + bs_ref[...]
    pred_ref[...] = pred
    d = pred - y_ref[...]
    i = pl.program_id(0)

    @pl.when(i == 0)
    def _():
        ls_ref[0, 0] = 0.0
        ps_ref[0, 0] = 0.0

    ls_ref[0, 0] += jnp.sum(d * d)
    ps_ref[0, 0] += jnp.sum(cnt_ref[...])


def kernel(mlp_input, mlp_output, W_enc, b_enc, W_dec, b_dec, W_skip, b_skip):
    be = b_enc.reshape(1, _H)
    bd = b_dec.reshape(1, _DOUT)
    bs = b_skip.reshape(1, _DOUT)

    pre = pl.pallas_call(
        _enc_body,
        grid=(_H // _BH_ENC,),
        in_specs=[
            pl.BlockSpec((_N, _DIN), lambda h: (0, 0)),
            pl.BlockSpec((_BH_ENC, _DIN), lambda h: (h, 0)),
            pl.BlockSpec((1, _BH_ENC), lambda h: (0, h)),
        ],
        out_specs=pl.BlockSpec((_N, _BH_ENC), lambda h: (0, h)),
        out_shape=jax.ShapeDtypeStruct((_N, _H), jnp.float32),
    )(mlp_input, W_enc, be)

    hidden, thr, cnt = pl.pallas_call(
        _topk_body,
        grid=(_N // _BN_TOP,),
        in_specs=[pl.BlockSpec((_BN_TOP, _H), lambda n: (n, 0))],
        out_specs=[
            pl.BlockSpec((_BN_TOP, _H), lambda n: (n, 0)),
            pl.BlockSpec((_BN_TOP, 1), lambda n: (n, 0)),
            pl.BlockSpec((_BN_TOP, 1), lambda n: (n, 0)),
        ],
        out_shape=[
            jax.ShapeDtypeStruct((_N, _H), jnp.float32),
            jax.ShapeDtypeStruct((_N, 1), jnp.float32),
            jax.ShapeDtypeStruct((_N, 1), jnp.float32),
        ],
    )(pre)
    del thr

    sp = pl.pallas_call(
        _dec_body,
        grid=(_H // _BH_DEC,),
        in_specs=[
            pl.BlockSpec((_N, _BH_DEC), lambda h: (0, h)),
            pl.BlockSpec((_DOUT, _BH_DEC), lambda h: (0, h)),
        ],
        out_specs=pl.BlockSpec((_N, _DOUT), lambda h: (0, 0)),
        out_shape=jax.ShapeDtypeStruct((_N, _DOUT), jnp.float32),
    )(hidden, W_dec)

    predicted, ls, ps = pl.pallas_call(
        _fin_body,
        grid=(_N // _BN_FIN,),
        in_specs=[
            pl.BlockSpec((_BN_FIN, _DOUT), lambda n: (n, 0)),
            pl.BlockSpec((_BN_FIN, _DIN), lambda n: (n, 0)),
            pl.BlockSpec((_DOUT, _DIN), lambda n: (0, 0)),
            pl.BlockSpec((1, _DOUT), lambda n: (0, 0)),
            pl.BlockSpec((1, _DOUT), lambda n: (0, 0)),
            pl.BlockSpec((_BN_FIN, _DOUT), lambda n: (n, 0)),
            pl.BlockSpec((_BN_FIN, 1), lambda n: (n, 0)),
        ],
        out_specs=[
            pl.BlockSpec((_BN_FIN, _DOUT), lambda n: (n, 0)),
            pl.BlockSpec(memory_space=pltpu.SMEM, block_shape=(1, 1),
                         index_map=lambda n: (0, 0)),
            pl.BlockSpec(memory_space=pltpu.SMEM, block_shape=(1, 1),
                         index_map=lambda n: (0, 0)),
        ],
        out_shape=[
            jax.ShapeDtypeStruct((_N, _DOUT), jnp.float32),
            jax.ShapeDtypeStruct((1, 1), jnp.float32),
            jax.ShapeDtypeStruct((1, 1), jnp.float32),
        ],
    )(sp, mlp_input, W_skip, bd, bs, mlp_output, cnt)

    recon = ls[0, 0] / jnp.float32(_N * _DOUT)
    l0 = ps[0, 0] / jnp.float32(_N)
    sparsity = jnp.float32(0.0)
    return predicted, hidden, recon, recon, sparsity, l0


# speedup vs baseline: 1.1823x; 1.1823x over previous
import jax
import jax.numpy as jnp
from jax.experimental import pallas as pl
from jax.experimental.pallas import tpu as pltpu

_N, _DIN, _DOUT, _H, _K = 2048, 2048, 2048, 16384, 32

_BH_ENC = 512
_BN_TOP = 128
_BH_DEC = 512
_BN_FIN = 256
_BISECT_ITERS = 36


def _enc_body(x_ref, w_ref, b_ref, out_ref):
    out_ref[...] = jax.lax.dot_general(
        x_ref[...], w_ref[...], (((1,), (1,)), ((), ())),
        preferred_element_type=jnp.float32) + b_ref[...]


def _topk_body(pre_ref, hid_ref, thr_ref, cnt_ref):
    v = pre_ref[...]
    kf = jnp.float32(_K)

    hi0 = jnp.max(v, axis=1, keepdims=True)
    clo = jnp.min(v, axis=1, keepdims=True)

    def cond(state):
        i, lo, hi, cnt = state
        return jnp.logical_and(i < _BISECT_ITERS,
                               jnp.logical_not(jnp.all(cnt == kf)))

    def body(state):
        i, lo, hi, cnt = state
        for _ in range(4):
            mid = (lo + hi) * 0.5
            c = jnp.sum((v >= mid).astype(jnp.float32), axis=1, keepdims=True)
            ge = c >= kf
            lo = jnp.where(ge, mid, lo)
            hi = jnp.where(ge, hi, mid)
            cnt = jnp.where(ge, c, cnt)
        return (i + 4, lo, hi, cnt)

    _, lo, hi, _ = jax.lax.while_loop(
        cond, body, (jnp.int32(0), clo, hi0, jnp.full_like(hi0, 1e9)))
    t = lo
    hid = jnp.where(v >= t, jnp.maximum(v, 0.0), 0.0)
    hid_ref[...] = hid
    thr_ref[...] = t
    cnt_ref[...] = jnp.sum((hid > 0).astype(jnp.float32), axis=1,
                           keepdims=True)


def _dec_body(hid_ref, w_ref, out_ref):
    h = pl.program_id(0)

    @pl.when(h == 0)
    def _():
        out_ref[...] = jnp.zeros_like(out_ref)

    out_ref[...] += jax.lax.dot_general(
        hid_ref[...].astype(jnp.bfloat16), w_ref[...].astype(jnp.bfloat16),
        (((1,), (1,)), ((), ())), preferred_element_type=jnp.float32)


def _fin_body(sp_ref, x_ref, ws_ref, bd_ref, bs_ref, y_ref, cnt_ref,
              pred_ref, ls_ref, ps_ref):
    skip = jax.lax.dot_general(
        x_ref[...].astype(jnp.bfloat16), ws_ref[...].astype(jnp.bfloat16),
        (((1,), (1,)), ((), ())), preferred_element_type=jnp.float32)
    pred = sp_ref[...] + skip + bd_ref[...] + bs_ref[...]
    pred_ref[...] = pred
    d = pred - y_ref[...]
    i = pl.program_id(0)

    @pl.when(i == 0)
    def _():
        ls_ref[0, 0] = 0.0
        ps_ref[0, 0] = 0.0

    ls_ref[0, 0] += jnp.sum(d * d)
    ps_ref[0, 0] += jnp.sum(cnt_ref[...])


def kernel(mlp_input, mlp_output, W_enc, b_enc, W_dec, b_dec, W_skip, b_skip):
    be = b_enc.reshape(1, _H)
    bd = b_dec.reshape(1, _DOUT)
    bs = b_skip.reshape(1, _DOUT)

    pre = pl.pallas_call(
        _enc_body,
        grid=(_H // _BH_ENC,),
        in_specs=[
            pl.BlockSpec((_N, _DIN), lambda h: (0, 0)),
            pl.BlockSpec((_BH_ENC, _DIN), lambda h: (h, 0)),
            pl.BlockSpec((1, _BH_ENC), lambda h: (0, h)),
        ],
        out_specs=pl.BlockSpec((_N, _BH_ENC), lambda h: (0, h)),
        out_shape=jax.ShapeDtypeStruct((_N, _H), jnp.float32),
    )(mlp_input, W_enc, be)

    hidden, thr, cnt = pl.pallas_call(
        _topk_body,
        grid=(_N // _BN_TOP,),
        in_specs=[pl.BlockSpec((_BN_TOP, _H), lambda n: (n, 0))],
        out_specs=[
            pl.BlockSpec((_BN_TOP, _H), lambda n: (n, 0)),
            pl.BlockSpec((_BN_TOP, 1), lambda n: (n, 0)),
            pl.BlockSpec((_BN_TOP, 1), lambda n: (n, 0)),
        ],
        out_shape=[
            jax.ShapeDtypeStruct((_N, _H), jnp.float32),
            jax.ShapeDtypeStruct((_N, 1), jnp.float32),
            jax.ShapeDtypeStruct((_N, 1), jnp.float32),
        ],
    )(pre)
    del thr

    sp = pl.pallas_call(
        _dec_body,
        grid=(_H // _BH_DEC,),
        in_specs=[
            pl.BlockSpec((_N, _BH_DEC), lambda h: (0, h)),
            pl.BlockSpec((_DOUT, _BH_DEC), lambda h: (0, h)),
        ],
        out_specs=pl.BlockSpec((_N, _DOUT), lambda h: (0, 0)),
        out_shape=jax.ShapeDtypeStruct((_N, _DOUT), jnp.float32),
    )(hidden, W_dec)

    predicted, ls, ps = pl.pallas_call(
        _fin_body,
        grid=(_N // _BN_FIN,),
        in_specs=[
            pl.BlockSpec((_BN_FIN, _DOUT), lambda n: (n, 0)),
            pl.BlockSpec((_BN_FIN, _DIN), lambda n: (n, 0)),
            pl.BlockSpec((_DOUT, _DIN), lambda n: (0, 0)),
            pl.BlockSpec((1, _DOUT), lambda n: (0, 0)),
            pl.BlockSpec((1, _DOUT), lambda n: (0, 0)),
            pl.BlockSpec((_BN_FIN, _DOUT), lambda n: (n, 0)),
            pl.BlockSpec((_BN_FIN, 1), lambda n: (n, 0)),
        ],
        out_specs=[
            pl.BlockSpec((_BN_FIN, _DOUT), lambda n: (n, 0)),
            pl.BlockSpec(memory_space=pltpu.SMEM, block_shape=(1, 1),
                         index_map=lambda n: (0, 0)),
            pl.BlockSpec(memory_space=pltpu.SMEM, block_shape=(1, 1),
                         index_map=lambda n: (0, 0)),
        ],
        out_shape=[
            jax.ShapeDtypeStruct((_N, _DOUT), jnp.float32),
            jax.ShapeDtypeStruct((1, 1), jnp.float32),
            jax.ShapeDtypeStruct((1, 1), jnp.float32),
        ],
    )(sp, mlp_input, W_skip, bd, bs, mlp_output, cnt)

    recon = ls[0, 0] / jnp.float32(_N * _DOUT)
    l0 = ps[0, 0] / jnp.float32(_N)
    sparsity = jnp.float32(0.0)
    return predicted, hidden, recon, recon, sparsity, l0


# speedup vs baseline: 1.1840x; 1.0014x over previous
import jax
import jax.numpy as jnp
from jax.experimental import pallas as pl
from jax.experimental.pallas import tpu as pltpu

_N, _DIN, _DOUT, _H, _K = 2048, 2048, 2048, 16384, 32

_BH_ENC = 512
_BN_TOP = 128
_BH_DEC = 512
_BN_FIN = 256
_BISECT_ITERS = 36


def _enc_body(x_ref, w_ref, b_ref, out_ref):
    out_ref[...] = jax.lax.dot_general(
        x_ref[...], w_ref[...], (((1,), (1,)), ((), ())),
        preferred_element_type=jnp.float32) + b_ref[...]


def _topk_body(pre_ref, hid_ref, thr_ref, cnt_ref):
    v = pre_ref[...]
    kf = jnp.float32(_K)

    hi0 = jnp.max(v, axis=1, keepdims=True)
    clo = jnp.min(v, axis=1, keepdims=True)

    def cond(state):
        i, lo, hi, cnt = state
        return jnp.logical_and(i < _BISECT_ITERS,
                               jnp.logical_not(jnp.all(cnt == kf)))

    def body(state):
        i, lo, hi, cnt = state
        for _ in range(4):
            mid = (lo + hi) * 0.5
            c = jnp.sum((v >= mid).astype(jnp.float32), axis=1, keepdims=True)
            ge = c >= kf
            lo = jnp.where(ge, mid, lo)
            hi = jnp.where(ge, hi, mid)
            cnt = jnp.where(ge, c, cnt)
        return (i + 4, lo, hi, cnt)

    _, lo, hi, _ = jax.lax.while_loop(
        cond, body, (jnp.int32(0), clo, hi0, jnp.full_like(hi0, 1e9)))
    t = lo
    hid = jnp.where(v >= t, jnp.maximum(v, 0.0), 0.0)
    hid_ref[...] = hid
    thr_ref[...] = t
    cnt_ref[...] = jnp.sum((hid > 0).astype(jnp.float32), axis=1,
                           keepdims=True)


def _dec_body(hid_ref, w_ref, out_ref):
    h = pl.program_id(0)

    @pl.when(h == 0)
    def _():
        out_ref[...] = jnp.zeros_like(out_ref)

    out_ref[...] += jax.lax.dot_general(
        hid_ref[...], w_ref[...], (((1,), (1,)), ((), ())),
        preferred_element_type=jnp.float32)


def _fin_body(sp_ref, x_ref, ws_ref, bd_ref, bs_ref, y_ref, cnt_ref,
              pred_ref, ls_ref, ps_ref):
    skip = jax.lax.dot_general(
        x_ref[...], ws_ref[...], (((1,), (1,)), ((), ())),
        preferred_element_type=jnp.float32)
    pred = sp_ref[...] + skip + bd_ref[...] + bs_ref[...]
    pred_ref[...] = pred
    d = pred - y_ref[...]
    i = pl.program_id(0)

    @pl.when(i == 0)
    def _():
        ls_ref[0, 0] = 0.0
        ps_ref[0, 0] = 0.0

    ls_ref[0, 0] += jnp.sum(d * d)
    ps_ref[0, 0] += jnp.sum(cnt_ref[...])


def kernel(mlp_input, mlp_output, W_enc, b_enc, W_dec, b_dec, W_skip, b_skip):
    be = b_enc.reshape(1, _H)
    bd = b_dec.reshape(1, _DOUT)
    bs = b_skip.reshape(1, _DOUT)

    pre = pl.pallas_call(
        _enc_body,
        grid=(_H // _BH_ENC,),
        in_specs=[
            pl.BlockSpec((_N, _DIN), lambda h: (0, 0)),
            pl.BlockSpec((_BH_ENC, _DIN), lambda h: (h, 0)),
            pl.BlockSpec((1, _BH_ENC), lambda h: (0, h)),
        ],
        out_specs=pl.BlockSpec((_N, _BH_ENC), lambda h: (0, h)),
        out_shape=jax.ShapeDtypeStruct((_N, _H), jnp.float32),
    )(mlp_input, W_enc, be)

    hidden, thr, cnt = pl.pallas_call(
        _topk_body,
        grid=(_N // _BN_TOP,),
        in_specs=[pl.BlockSpec((_BN_TOP, _H), lambda n: (n, 0))],
        out_specs=[
            pl.BlockSpec((_BN_TOP, _H), lambda n: (n, 0)),
            pl.BlockSpec((_BN_TOP, 1), lambda n: (n, 0)),
            pl.BlockSpec((_BN_TOP, 1), lambda n: (n, 0)),
        ],
        out_shape=[
            jax.ShapeDtypeStruct((_N, _H), jnp.float32),
            jax.ShapeDtypeStruct((_N, 1), jnp.float32),
            jax.ShapeDtypeStruct((_N, 1), jnp.float32),
        ],
    )(pre)
    del thr

    sp = pl.pallas_call(
        _dec_body,
        grid=(_H // _BH_DEC,),
        in_specs=[
            pl.BlockSpec((_N, _BH_DEC), lambda h: (0, h)),
            pl.BlockSpec((_DOUT, _BH_DEC), lambda h: (0, h)),
        ],
        out_specs=pl.BlockSpec((_N, _DOUT), lambda h: (0, 0)),
        out_shape=jax.ShapeDtypeStruct((_N, _DOUT), jnp.float32),
    )(hidden, W_dec)

    predicted, ls, ps = pl.pallas_call(
        _fin_body,
        grid=(_N // _BN_FIN,),
        in_specs=[
            pl.BlockSpec((_BN_FIN, _DOUT), lambda n: (n, 0)),
            pl.BlockSpec((_BN_FIN, _DIN), lambda n: (n, 0)),
            pl.BlockSpec((_DOUT, _DIN), lambda n: (0, 0)),
            pl.BlockSpec((1, _DOUT), lambda n: (0, 0)),
            pl.BlockSpec((1, _DOUT), lambda n: (0, 0)),
            pl.BlockSpec((_BN_FIN, _DOUT), lambda n: (n, 0)),
            pl.BlockSpec((_BN_FIN, 1), lambda n: (n, 0)),
        ],
        out_specs=[
            pl.BlockSpec((_BN_FIN, _DOUT), lambda n: (n, 0)),
            pl.BlockSpec(memory_space=pltpu.SMEM, block_shape=(1, 1),
                         index_map=lambda n: (0, 0)),
            pl.BlockSpec(memory_space=pltpu.SMEM, block_shape=(1, 1),
                         index_map=lambda n: (0, 0)),
        ],
        out_shape=[
            jax.ShapeDtypeStruct((_N, _DOUT), jnp.float32),
            jax.ShapeDtypeStruct((1, 1), jnp.float32),
            jax.ShapeDtypeStruct((1, 1), jnp.float32),
        ],
    )(sp, mlp_input, W_skip, bd, bs, mlp_output, cnt)

    recon = ls[0, 0] / jnp.float32(_N * _DOUT)
    l0 = ps[0, 0] / jnp.float32(_N)
    sparsity = jnp.float32(0.0)
    return predicted, hidden, recon, recon, sparsity, l0


# speedup vs baseline: 1.1862x; 1.0019x over previous
import jax
import jax.numpy as jnp
from jax.experimental import pallas as pl
from jax.experimental.pallas import tpu as pltpu

_N, _DIN, _DOUT, _H, _K = 2048, 2048, 2048, 16384, 32

_BH_ENC = 512
_BN_TOP = 128
_BH_DEC = 512
_BN_FIN = 256
_BISECT_ITERS = 36


def _enc_body(x_ref, w_ref, b_ref, out_ref):
    out_ref[...] = jax.lax.dot_general(
        x_ref[...], w_ref[...], (((1,), (1,)), ((), ())),
        preferred_element_type=jnp.float32) + b_ref[...]


def _topk_body(pre_ref, hid_ref, cnt_ref):
    v = pre_ref[...]
    kf = jnp.float32(_K)

    hi0 = jnp.max(v, axis=1, keepdims=True)
    clo = jnp.min(v, axis=1, keepdims=True)

    def cond(state):
        i, lo, hi, cnt = state
        return jnp.logical_and(i < _BISECT_ITERS,
                               jnp.logical_not(jnp.all(cnt == kf)))

    def body(state):
        i, lo, hi, cnt = state
        for _ in range(4):
            mid = (lo + hi) * 0.5
            c = jnp.sum((v >= mid).astype(jnp.float32), axis=1, keepdims=True)
            ge = c >= kf
            lo = jnp.where(ge, mid, lo)
            hi = jnp.where(ge, hi, mid)
            cnt = jnp.where(ge, c, cnt)
        return (i + 4, lo, hi, cnt)

    _, lo, hi, _ = jax.lax.while_loop(
        cond, body, (jnp.int32(0), clo, hi0, jnp.full_like(hi0, 1e9)))
    t = lo
    hid = jnp.where(v >= t, jnp.maximum(v, 0.0), 0.0)
    hid_ref[...] = hid
    cnt_ref[...] = jnp.sum((hid > 0).astype(jnp.float32), axis=1,
                           keepdims=True)


def _dec_body(hid_ref, w_ref, out_ref):
    h = pl.program_id(0)

    @pl.when(h == 0)
    def _():
        out_ref[...] = jnp.zeros_like(out_ref)

    out_ref[...] += jax.lax.dot_general(
        hid_ref[...], w_ref[...], (((1,), (1,)), ((), ())),
        preferred_element_type=jnp.float32)


def _fin_body(sp_ref, x_ref, ws_ref, bd_ref, bs_ref, y_ref, cnt_ref,
              pred_ref, ls_ref, ps_ref):
    skip = jax.lax.dot_general(
        x_ref[...], ws_ref[...], (((1,), (1,)), ((), ())),
        preferred_element_type=jnp.float32)
    pred = sp_ref[...] + skip + bd_ref[...] + bs_ref[...]
    pred_ref[...] = pred
    d = pred - y_ref[...]
    i = pl.program_id(0)

    @pl.when(i == 0)
    def _():
        ls_ref[0, 0] = 0.0
        ps_ref[0, 0] = 0.0

    ls_ref[0, 0] += jnp.sum(d * d)
    ps_ref[0, 0] += jnp.sum(cnt_ref[...])


def kernel(mlp_input, mlp_output, W_enc, b_enc, W_dec, b_dec, W_skip, b_skip):
    be = b_enc.reshape(1, _H)
    bd = b_dec.reshape(1, _DOUT)
    bs = b_skip.reshape(1, _DOUT)

    pre = pl.pallas_call(
        _enc_body,
        grid=(_H // _BH_ENC,),
        in_specs=[
            pl.BlockSpec((_N, _DIN), lambda h: (0, 0)),
            pl.BlockSpec((_BH_ENC, _DIN), lambda h: (h, 0)),
            pl.BlockSpec((1, _BH_ENC), lambda h: (0, h)),
        ],
        out_specs=pl.BlockSpec((_N, _BH_ENC), lambda h: (0, h)),
        out_shape=jax.ShapeDtypeStruct((_N, _H), jnp.float32),
    )(mlp_input, W_enc, be)

    hidden, cnt = pl.pallas_call(
        _topk_body,
        grid=(_N // _BN_TOP,),
        in_specs=[pl.BlockSpec((_BN_TOP, _H), lambda n: (n, 0))],
        out_specs=[
            pl.BlockSpec((_BN_TOP, _H), lambda n: (n, 0)),
            pl.BlockSpec((_BN_TOP, 1), lambda n: (n, 0)),
        ],
        out_shape=[
            jax.ShapeDtypeStruct((_N, _H), jnp.float32),
            jax.ShapeDtypeStruct((_N, 1), jnp.float32),
        ],
    )(pre)

    sp = pl.pallas_call(
        _dec_body,
        grid=(_H // _BH_DEC,),
        in_specs=[
            pl.BlockSpec((_N, _BH_DEC), lambda h: (0, h)),
            pl.BlockSpec((_DOUT, _BH_DEC), lambda h: (0, h)),
        ],
        out_specs=pl.BlockSpec((_N, _DOUT), lambda h: (0, 0)),
        out_shape=jax.ShapeDtypeStruct((_N, _DOUT), jnp.float32),
    )(hidden, W_dec)

    predicted, ls, ps = pl.pallas_call(
        _fin_body,
        grid=(_N // _BN_FIN,),
        in_specs=[
            pl.BlockSpec((_BN_FIN, _DOUT), lambda n: (n, 0)),
            pl.BlockSpec((_BN_FIN, _DIN), lambda n: (n, 0)),
            pl.BlockSpec((_DOUT, _DIN), lambda n: (0, 0)),
            pl.BlockSpec((1, _DOUT), lambda n: (0, 0)),
            pl.BlockSpec((1, _DOUT), lambda n: (0, 0)),
            pl.BlockSpec((_BN_FIN, _DOUT), lambda n: (n, 0)),
            pl.BlockSpec((_BN_FIN, 1), lambda n: (n, 0)),
        ],
        out_specs=[
            pl.BlockSpec((_BN_FIN, _DOUT), lambda n: (n, 0)),
            pl.BlockSpec(memory_space=pltpu.SMEM, block_shape=(1, 1),
                         index_map=lambda n: (0, 0)),
            pl.BlockSpec(memory_space=pltpu.SMEM, block_shape=(1, 1),
                         index_map=lambda n: (0, 0)),
        ],
        out_shape=[
            jax.ShapeDtypeStruct((_N, _DOUT), jnp.float32),
            jax.ShapeDtypeStruct((1, 1), jnp.float32),
            jax.ShapeDtypeStruct((1, 1), jnp.float32),
        ],
    )(sp, mlp_input, W_skip, bd, bs, mlp_output, cnt)

    recon = ls[0, 0] / jnp.float32(_N * _DOUT)
    l0 = ps[0, 0] / jnp.float32(_N)
    sparsity = jnp.float32(0.0)
    return predicted, hidden, recon, recon, sparsity, l0
